# baseline (speedup 1.0000x reference)
import functools

import jax
import jax.numpy as jnp
from jax import lax
from jax.experimental import pallas as pl
from jax.experimental.pallas import tpu as pltpu
from jax.experimental.pallas import tpu_sc as plsc

OUT = 32
BIG = 3.0e38
ROW_BLK = 2000

NC, NS, L = 2, 16, 16
NW = NC * NS
MAXR = 6800
EDGE_CHUNK = 1600


def _embed_body(x_ref, w_ref, b_ref, o_ref):
    y = jnp.dot(x_ref[...], w_ref[...], preferred_element_type=jnp.float32)
    y = y + b_ref[...]
    o_ref[...] = jnp.maximum(y, 0.01 * y)


def _embed(x, W, b):
    n, k = x.shape
    kp = 8
    x = jnp.pad(x, ((0, 0), (0, kp - k)))
    W = jnp.pad(W, ((0, kp - k), (0, 0)))
    grid = n // ROW_BLK
    return pl.pallas_call(
        _embed_body,
        grid=(grid,),
        in_specs=[
            pl.BlockSpec((ROW_BLK, kp), lambda i: (i, 0)),
            pl.BlockSpec((kp, OUT), lambda i: (0, 0)),
            pl.BlockSpec((1, OUT), lambda i: (0, 0)),
        ],
        out_specs=pl.BlockSpec((ROW_BLK, OUT), lambda i: (i, 0)),
        out_shape=jax.ShapeDtypeStruct((n, OUT), jnp.float32),
    )(x, W, b.reshape(1, OUT))


def _finalize_body(xd_ref, m_ref, w1_ref, w2_ref, b_ref, o_ref):
    xd = xd_ref[...]
    m = m_ref[...].astype(jnp.float32)
    flag = m[:, 0:1] < 1e38
    mx = jnp.where(flag, xd - m, 0.0)
    y = jnp.dot(xd, w1_ref[...], preferred_element_type=jnp.float32)
    y = y + jnp.dot(mx, w2_ref[...], preferred_element_type=jnp.float32)
    y = y + b_ref[...]
    y = jnp.maximum(y, 0.01 * y)
    o_ref[...] = xd + y


def _finalize(x_dst, m, W, b):
    n = x_dst.shape[0]
    grid = n // ROW_BLK
    return pl.pallas_call(
        _finalize_body,
        grid=(grid,),
        in_specs=[
            pl.BlockSpec((ROW_BLK, OUT), lambda i: (i, 0)),
            pl.BlockSpec((ROW_BLK, OUT), lambda i: (i, 0)),
            pl.BlockSpec((OUT, OUT), lambda i: (0, 0)),
            pl.BlockSpec((OUT, OUT), lambda i: (0, 0)),
            pl.BlockSpec((1, OUT), lambda i: (0, 0)),
        ],
        out_specs=pl.BlockSpec((ROW_BLK, OUT), lambda i: (i, 0)),
        out_shape=jax.ShapeDtypeStruct((n, OUT), jnp.float32),
    )(x_dst, m, W[:OUT], W[OUT:], b.reshape(1, OUT))


@functools.partial(jax.jit, static_argnums=(3,))
def _segmin_sc_call(x_src, src, dst, n_dst):
    m_edges = src.shape[0]
    npass = -(-n_dst // (NW * MAXR))
    R = -(-n_dst // (NW * npass))
    R = -(-R // 8) * 8
    npad = NW * npass * R
    C = EDGE_CHUNK
    nchunk = -(-m_edges // C)
    if nchunk % 2:
        nchunk += 1
    if nchunk * C != m_edges:
        pad = nchunk * C - m_edges
        src = jnp.concatenate([src, jnp.zeros((pad,), jnp.int32)])
        dst = jnp.concatenate([dst, jnp.full((pad,), npad, jnp.int32)])
    npair = nchunk // 2

    mesh = plsc.VectorSubcoreMesh(core_axis_name="c", subcore_axis_name="s")

    def body(xsrc, srch, dsth, outh,
             accf, dstvA, srcvA, dstvB, srcvB, dloc, ssel,
             idx0, idx1, idx2, idx3, rows0, rows1, rows2, rows3,
             semA, semB, semg0, semg1, semg2, semg3):
        cix = lax.axis_index("c")
        six = lax.axis_index("s")
        wid = six * NC + cix

        def load_chunk(ci, dv, sv, sem):
            pltpu.async_copy(dsth.at[pl.ds(ci * C, C)], dv, sem)
            pltpu.async_copy(srch.at[pl.ds(ci * C, C)], sv, sem)

        def wait_chunk(ci, dv, sv, sem):
            pltpu.make_async_copy(dsth.at[pl.ds(ci * C, C)], dv, sem).wait()
            pltpu.make_async_copy(srch.at[pl.ds(ci * C, C)], sv, sem).wait()

        def start_gather(j, idxr, rowsr, semg):
            idxr[pl.ds(0, L)] = ssel[pl.ds(j * 2 * L, L)]
            idxr[pl.ds(L, L)] = ssel[pl.ds(j * 2 * L + L, L)]
            pltpu.async_copy(xsrc.at[idxr], rowsr, semg)

        def wait_gather(idxr, rowsr, semg):
            pltpu.make_async_copy(xsrc.at[idxr], rowsr, semg).wait()

        def accum(j, rowsr):
            for h in range(2):
                dlv = dloc[pl.ds(j * 2 * L + h * L, L)]
                for e in range(L):
                    dle = dlv[e]
                    g = rowsr[h * L + e, pl.ds(0, OUT)]
                    accf[dle, pl.ds(0, OUT)] = jnp.minimum(
                        accf[dle, pl.ds(0, OUT)], g)

        def pass_body(p, _):
            rowbase = (wid * npass + p) * R
            load_chunk(0, dstvA, srcvA, semA)

            def init_body(i, _):
                big = jnp.full((OUT,), BIG, jnp.bfloat16)
                for q in range(4):
                    accf[4 * i + q, pl.ds(0, OUT)] = big
                return 0

            lax.fori_loop(0, (R + 8) // 4, init_body, 0)

            def process(ci, dv, sv):
                def group_body(g2, cnt):
                    for u in range(2):
                        g = 2 * g2 + u
                        d = dv[pl.ds(g * L, L)]
                        sidx = sv[pl.ds(g * L, L)]
                        dl = d - rowbase
                        mask = (dl >= 0) & (dl < R)
                        plsc.store_compressed(dloc.at[pl.ds(cnt, L)], dl, mask=mask)
                        plsc.store_compressed(ssel.at[pl.ds(cnt, L)], sidx, mask=mask)
                        cnt = cnt + plsc.all_reduce_population_count(mask)[0]
                    return cnt

                cnt = lax.fori_loop(0, C // L // 2, group_body, 0)
                full_true = jnp.ones((L,), jnp.bool_)
                for h in range(2):
                    plsc.store_compressed(dloc.at[pl.ds(cnt + h * L, L)],
                                          jnp.full((L,), R, jnp.int32),
                                          mask=full_true)
                    plsc.store_compressed(ssel.at[pl.ds(cnt + h * L, L)],
                                          jnp.zeros((L,), jnp.int32),
                                          mask=full_true)
                ngr = (cnt + (2 * L - 1)) // (2 * L)
                bufs = ((idx0, rows0, semg0), (idx1, rows1, semg1),
                        (idx2, rows2, semg2), (idx3, rows3, semg3))

                for q in range(4):

                    @pl.when(q < ngr)
                    def _(q=q):
                        start_gather(q, *bufs[q])

                def quad_body(tq, _):
                    for q in range(4):
                        j = 4 * tq + q

                        @pl.when(j < ngr)
                        def _(j=j, q=q):
                            idxr, rowsr, semg = bufs[q]
                            wait_gather(idxr, rowsr, semg)
                            accum(j, rowsr)

                            @pl.when(j + 4 < ngr)
                            def _(j=j, q=q):
                                start_gather(j + 4, *bufs[q])

                    return 0

                lax.fori_loop(0, (ngr + 3) // 4, quad_body, 0)

            def pair_chunk(t, _):
                ci0 = 2 * t
                ci1 = ci0 + 1
                load_chunk(ci1, dstvB, srcvB, semB)
                wait_chunk(ci0, dstvA, srcvA, semA)
                process(ci0, dstvA, srcvA)

                @pl.when(ci0 + 2 < nchunk)
                def _():
                    load_chunk(ci0 + 2, dstvA, srcvA, semA)

                wait_chunk(ci1, dstvB, srcvB, semB)
                process(ci1, dstvB, srcvB)
                return 0

            lax.fori_loop(0, npair, pair_chunk, 0)
            pltpu.sync_copy(accf.at[pl.ds(0, R)], outh.at[pl.ds(rowbase, R)])
            return 0

        lax.fori_loop(0, npass, pass_body, 0)

    f = pl.kernel(
        body,
        out_type=jax.ShapeDtypeStruct((npad, OUT), jnp.bfloat16),
        mesh=mesh,
        scratch_types=[
            pltpu.VMEM((R + 8, OUT), jnp.bfloat16),
            pltpu.VMEM((C,), jnp.int32),
            pltpu.VMEM((C,), jnp.int32),
            pltpu.VMEM((C,), jnp.int32),
            pltpu.VMEM((C,), jnp.int32),
            pltpu.VMEM((C + 2 * L,), jnp.int32),
            pltpu.VMEM((C + 2 * L,), jnp.int32),
            pltpu.VMEM((2 * L,), jnp.int32),
            pltpu.VMEM((2 * L,), jnp.int32),
            pltpu.VMEM((2 * L,), jnp.int32),
            pltpu.VMEM((2 * L,), jnp.int32),
            pltpu.VMEM((2 * L, OUT), jnp.bfloat16),
            pltpu.VMEM((2 * L, OUT), jnp.bfloat16),
            pltpu.VMEM((2 * L, OUT), jnp.bfloat16),
            pltpu.VMEM((2 * L, OUT), jnp.bfloat16),
            pltpu.SemaphoreType.DMA,
            pltpu.SemaphoreType.DMA,
            pltpu.SemaphoreType.DMA,
            pltpu.SemaphoreType.DMA,
            pltpu.SemaphoreType.DMA,
            pltpu.SemaphoreType.DMA,
        ],
        compiler_params=pltpu.CompilerParams(
            needs_layout_passes=False, use_tc_tiling_on_sc=False),
    )
    out = f(x_src.astype(jnp.bfloat16), src, dst)
    return out[:n_dst]


def _segmin(x_src, src, dst, n_dst):
    return _segmin_sc_call(x_src, src, dst, n_dst)


def _conv(x_src, x_dst, e, W, b):
    m = _segmin(x_src, e[0], e[1], x_dst.shape[0])
    return _finalize(x_dst, m, W, b)


def kernel(x_f, x_e, x_v, e_fe, e_ev, e_ef, e_ve, Wf, bf, We, be, Wv, bv, Wfe, bfe, Wev, bev):
    x_f = _embed(x_f, Wf, bf)
    x_e = _embed(x_e, We, be)
    x_v = _embed(x_v, Wv, bv)
    x_e = _conv(x_f, x_e, e_fe, Wfe, bfe)
    x_v = _conv(x_e, x_v, e_ev, Wev, bev)
    x_f = _conv(x_e, x_f, e_ef, Wfe, bfe)
    x_e = _conv(x_v, x_e, e_ve, Wfe, bfe)
    return (x_f, x_e, x_v)


# speedup vs baseline: 1.2697x; 1.2697x over previous
import functools

import jax
import jax.numpy as jnp
from jax import lax
from jax.experimental import pallas as pl
from jax.experimental.pallas import tpu as pltpu
from jax.experimental.pallas import tpu_sc as plsc

OUT = 32
BIG = 3.0e38
ROW_BLK = 2000

NC, NS, L = 2, 16, 16
NW = NC * NS
MAXR = 6800
EDGE_CHUNK = 3200


def _embed_body(x_ref, w_ref, b_ref, o_ref):
    y = jnp.dot(x_ref[...], w_ref[...], preferred_element_type=jnp.float32)
    y = y + b_ref[...]
    o_ref[...] = jnp.maximum(y, 0.01 * y)


def _embed_body2(x_ref, w_ref, b_ref, o_ref, o16_ref):
    y = jnp.dot(x_ref[...], w_ref[...], preferred_element_type=jnp.float32)
    y = y + b_ref[...]
    y = jnp.maximum(y, 0.01 * y)
    o_ref[...] = y
    o16_ref[...] = y.astype(jnp.bfloat16)


def _embed(x, W, b, want_bf16=False):
    n, k = x.shape
    kp = 8
    x = jnp.pad(x, ((0, 0), (0, kp - k)))
    W = jnp.pad(W, ((0, kp - k), (0, 0)))
    grid = n // ROW_BLK
    blk = pl.BlockSpec((ROW_BLK, OUT), lambda i: (i, 0))
    in_specs = [
        pl.BlockSpec((ROW_BLK, kp), lambda i: (i, 0)),
        pl.BlockSpec((kp, OUT), lambda i: (0, 0)),
        pl.BlockSpec((1, OUT), lambda i: (0, 0)),
    ]
    if not want_bf16:
        return pl.pallas_call(
            _embed_body,
            grid=(grid,),
            in_specs=in_specs,
            out_specs=blk,
            out_shape=jax.ShapeDtypeStruct((n, OUT), jnp.float32),
        )(x, W, b.reshape(1, OUT))
    return pl.pallas_call(
        _embed_body2,
        grid=(grid,),
        in_specs=in_specs,
        out_specs=(blk, blk),
        out_shape=(jax.ShapeDtypeStruct((n, OUT), jnp.float32),
                   jax.ShapeDtypeStruct((n, OUT), jnp.bfloat16)),
    )(x, W, b.reshape(1, OUT))


def _fin_y(xd, m_ref, w1_ref, w2_ref, b_ref):
    m = m_ref[...].astype(jnp.float32)
    flag = m[:, 0:1] < 1e38
    mx = jnp.where(flag, xd - m, 0.0)
    y = jnp.dot(xd, w1_ref[...], preferred_element_type=jnp.float32)
    y = y + jnp.dot(mx, w2_ref[...], preferred_element_type=jnp.float32)
    y = y + b_ref[...]
    return jnp.maximum(y, 0.01 * y)


def _finalize_body(xd_ref, m_ref, w1_ref, w2_ref, b_ref, o_ref):
    xd = xd_ref[...]
    o_ref[...] = xd + _fin_y(xd, m_ref, w1_ref, w2_ref, b_ref)


def _finalize_body2(xd_ref, m_ref, w1_ref, w2_ref, b_ref, o_ref, o16_ref):
    xd = xd_ref[...]
    o = xd + _fin_y(xd, m_ref, w1_ref, w2_ref, b_ref)
    o_ref[...] = o
    o16_ref[...] = o.astype(jnp.bfloat16)


def _finalize(x_dst, m, W, b, want_bf16=False):
    n = x_dst.shape[0]
    grid = n // ROW_BLK
    blk = pl.BlockSpec((ROW_BLK, OUT), lambda i: (i, 0))
    in_specs = [
        blk,
        blk,
        pl.BlockSpec((OUT, OUT), lambda i: (0, 0)),
        pl.BlockSpec((OUT, OUT), lambda i: (0, 0)),
        pl.BlockSpec((1, OUT), lambda i: (0, 0)),
    ]
    args = (x_dst, m, W[:OUT], W[OUT:], b.reshape(1, OUT))
    if not want_bf16:
        return pl.pallas_call(
            _finalize_body,
            grid=(grid,),
            in_specs=in_specs,
            out_specs=blk,
            out_shape=jax.ShapeDtypeStruct((n, OUT), jnp.float32),
        )(*args)
    return pl.pallas_call(
        _finalize_body2,
        grid=(grid,),
        in_specs=in_specs,
        out_specs=(blk, blk),
        out_shape=(jax.ShapeDtypeStruct((n, OUT), jnp.float32),
                   jax.ShapeDtypeStruct((n, OUT), jnp.bfloat16)),
    )(*args)


@functools.partial(jax.jit, static_argnums=(3,))
def _segmin_sc_call(x_src, src, dst, n_dst):
    m_edges = src.shape[0]
    npass = -(-n_dst // (NW * MAXR))
    R = -(-n_dst // (NW * npass))
    R = -(-R // 8) * 8
    npad = NW * npass * R
    C = EDGE_CHUNK
    nchunk = -(-m_edges // C)
    if nchunk % 2:
        nchunk += 1
    if nchunk * C != m_edges:
        pad = nchunk * C - m_edges
        src = jnp.concatenate([src, jnp.zeros((pad,), jnp.int32)])
        dst = jnp.concatenate([dst, jnp.full((pad,), npad, jnp.int32)])
    npair = nchunk // 2

    mesh = plsc.VectorSubcoreMesh(core_axis_name="c", subcore_axis_name="s")

    def body(xsrc, srch, dsth, outh,
             accf, dstvA, srcvA, dstvB, srcvB, dloc, ssel,
             idx0, idx1, idx2, idx3, rows0, rows1, rows2, rows3,
             semA, semB, semg0, semg1, semg2, semg3):
        cix = lax.axis_index("c")
        six = lax.axis_index("s")
        wid = six * NC + cix

        def load_chunk(ci, dv, sv, sem):
            pltpu.async_copy(dsth.at[pl.ds(ci * C, C)], dv, sem)
            pltpu.async_copy(srch.at[pl.ds(ci * C, C)], sv, sem)

        def wait_chunk(ci, dv, sv, sem):
            pltpu.make_async_copy(dsth.at[pl.ds(ci * C, C)], dv, sem).wait()
            pltpu.make_async_copy(srch.at[pl.ds(ci * C, C)], sv, sem).wait()

        def start_gather(j, idxr, rowsr, semg):
            idxr[pl.ds(0, L)] = ssel[pl.ds(j * L, L)]
            pltpu.async_copy(xsrc.at[idxr], rowsr, semg)

        def wait_gather(idxr, rowsr, semg):
            pltpu.make_async_copy(xsrc.at[idxr], rowsr, semg).wait()

        def accum(j, rowsr):
            dlv = dloc[pl.ds(j * L, L)]
            for e in range(L):
                dle = dlv[e]
                g = rowsr[e, pl.ds(0, OUT)]
                accf[dle, pl.ds(0, OUT)] = jnp.minimum(accf[dle, pl.ds(0, OUT)], g)

        def pass_body(p, _):
            rowbase = (wid * npass + p) * R
            load_chunk(0, dstvA, srcvA, semA)

            def init_body(i, _):
                big = jnp.full((OUT,), BIG, jnp.bfloat16)
                for q in range(4):
                    accf[4 * i + q, pl.ds(0, OUT)] = big
                return 0

            lax.fori_loop(0, (R + 8) // 4, init_body, 0)

            def process(ci, dv, sv):
                def group_body(g2, cnt):
                    for u in range(2):
                        g = 2 * g2 + u
                        d = dv[pl.ds(g * L, L)]
                        sidx = sv[pl.ds(g * L, L)]
                        dl = d - rowbase
                        mask = (dl >= 0) & (dl < R)
                        plsc.store_compressed(dloc.at[pl.ds(cnt, L)], dl, mask=mask)
                        plsc.store_compressed(ssel.at[pl.ds(cnt, L)], sidx, mask=mask)
                        cnt = cnt + plsc.all_reduce_population_count(mask)[0]
                    return cnt

                cnt = lax.fori_loop(0, C // L // 2, group_body, 0)
                full_true = jnp.ones((L,), jnp.bool_)
                plsc.store_compressed(dloc.at[pl.ds(cnt, L)],
                                      jnp.full((L,), R, jnp.int32), mask=full_true)
                plsc.store_compressed(ssel.at[pl.ds(cnt, L)],
                                      jnp.zeros((L,), jnp.int32), mask=full_true)
                ngr = (cnt + (L - 1)) // L
                bufs = ((idx0, rows0, semg0), (idx1, rows1, semg1),
                        (idx2, rows2, semg2), (idx3, rows3, semg3))

                for q in range(4):

                    @pl.when(q < ngr)
                    def _(q=q):
                        start_gather(q, *bufs[q])

                def quad_body(tq, _):
                    for q in range(4):
                        j = 4 * tq + q

                        @pl.when(j < ngr)
                        def _(j=j, q=q):
                            idxr, rowsr, semg = bufs[q]
                            wait_gather(idxr, rowsr, semg)
                            accum(j, rowsr)

                            @pl.when(j + 4 < ngr)
                            def _(j=j, q=q):
                                start_gather(j + 4, *bufs[q])

                    return 0

                lax.fori_loop(0, (ngr + 3) // 4, quad_body, 0)

            def pair_chunk(t, _):
                ci0 = 2 * t
                ci1 = ci0 + 1
                load_chunk(ci1, dstvB, srcvB, semB)
                wait_chunk(ci0, dstvA, srcvA, semA)
                process(ci0, dstvA, srcvA)

                @pl.when(ci0 + 2 < nchunk)
                def _():
                    load_chunk(ci0 + 2, dstvA, srcvA, semA)

                wait_chunk(ci1, dstvB, srcvB, semB)
                process(ci1, dstvB, srcvB)
                return 0

            lax.fori_loop(0, npair, pair_chunk, 0)
            pltpu.sync_copy(accf.at[pl.ds(0, R)], outh.at[pl.ds(rowbase, R)])
            return 0

        lax.fori_loop(0, npass, pass_body, 0)

    f = pl.kernel(
        body,
        out_type=jax.ShapeDtypeStruct((npad, OUT), jnp.bfloat16),
        mesh=mesh,
        scratch_types=[
            pltpu.VMEM((R + 8, OUT), jnp.bfloat16),
            pltpu.VMEM((C,), jnp.int32),
            pltpu.VMEM((C,), jnp.int32),
            pltpu.VMEM((C,), jnp.int32),
            pltpu.VMEM((C,), jnp.int32),
            pltpu.VMEM((C + L,), jnp.int32),
            pltpu.VMEM((C + L,), jnp.int32),
            pltpu.VMEM((L,), jnp.int32),
            pltpu.VMEM((L,), jnp.int32),
            pltpu.VMEM((L,), jnp.int32),
            pltpu.VMEM((L,), jnp.int32),
            pltpu.VMEM((L, OUT), jnp.bfloat16),
            pltpu.VMEM((L, OUT), jnp.bfloat16),
            pltpu.VMEM((L, OUT), jnp.bfloat16),
            pltpu.VMEM((L, OUT), jnp.bfloat16),
            pltpu.SemaphoreType.DMA,
            pltpu.SemaphoreType.DMA,
            pltpu.SemaphoreType.DMA,
            pltpu.SemaphoreType.DMA,
            pltpu.SemaphoreType.DMA,
            pltpu.SemaphoreType.DMA,
        ],
        compiler_params=pltpu.CompilerParams(
            needs_layout_passes=False, use_tc_tiling_on_sc=False),
    )
    out = f(x_src, src, dst)
    return out[:n_dst]


def _conv(x_src16, x_dst, e, W, b, want_bf16=False):
    m = _segmin_sc_call(x_src16, e[0], e[1], x_dst.shape[0])
    return _finalize(x_dst, m, W, b, want_bf16=want_bf16)


def kernel(x_f, x_e, x_v, e_fe, e_ev, e_ef, e_ve, Wf, bf, We, be, Wv, bv, Wfe, bfe, Wev, bev):
    x_f, x_f16 = _embed(x_f, Wf, bf, want_bf16=True)
    x_e = _embed(x_e, We, be)
    x_v = _embed(x_v, Wv, bv)
    x_e, x_e16 = _conv(x_f16, x_e, e_fe, Wfe, bfe, want_bf16=True)
    x_v, x_v16 = _conv(x_e16, x_v, e_ev, Wev, bev, want_bf16=True)
    x_f = _conv(x_e16, x_f, e_ef, Wfe, bfe)
    x_e = _conv(x_v16, x_e, e_ve, Wfe, bfe)
    return (x_f, x_e, x_v)


# speedup vs baseline: 1.3733x; 1.0815x over previous
import functools

import jax
import jax.numpy as jnp
from jax import lax
from jax.experimental import pallas as pl
from jax.experimental.pallas import tpu as pltpu
from jax.experimental.pallas import tpu_sc as plsc

OUT = 32
BIG = 3.0e38
ROW_BLK = 2000

NC, NS, L = 2, 16, 16
NW = NC * NS
MAXR = 6800
EDGE_CHUNK = 2400


def _embed_body(x_ref, w_ref, b_ref, o_ref):
    y = jnp.dot(x_ref[...], w_ref[...], preferred_element_type=jnp.float32)
    y = y + b_ref[...]
    o_ref[...] = jnp.maximum(y, 0.01 * y)


def _embed_body2(x_ref, w_ref, b_ref, o_ref, o16_ref):
    y = jnp.dot(x_ref[...], w_ref[...], preferred_element_type=jnp.float32)
    y = y + b_ref[...]
    y = jnp.maximum(y, 0.01 * y)
    o_ref[...] = y
    o16_ref[...] = y.astype(jnp.bfloat16)


def _embed(x, W, b, want_bf16=False):
    n, k = x.shape
    kp = 8
    x = jnp.pad(x, ((0, 0), (0, kp - k)))
    W = jnp.pad(W, ((0, kp - k), (0, 0)))
    grid = n // ROW_BLK
    blk = pl.BlockSpec((ROW_BLK, OUT), lambda i: (i, 0))
    in_specs = [
        pl.BlockSpec((ROW_BLK, kp), lambda i: (i, 0)),
        pl.BlockSpec((kp, OUT), lambda i: (0, 0)),
        pl.BlockSpec((1, OUT), lambda i: (0, 0)),
    ]
    if not want_bf16:
        return pl.pallas_call(
            _embed_body,
            grid=(grid,),
            in_specs=in_specs,
            out_specs=blk,
            out_shape=jax.ShapeDtypeStruct((n, OUT), jnp.float32),
        )(x, W, b.reshape(1, OUT))
    return pl.pallas_call(
        _embed_body2,
        grid=(grid,),
        in_specs=in_specs,
        out_specs=(blk, blk),
        out_shape=(jax.ShapeDtypeStruct((n, OUT), jnp.float32),
                   jax.ShapeDtypeStruct((n, OUT), jnp.bfloat16)),
    )(x, W, b.reshape(1, OUT))


def _fin_y(xd, m_ref, w1_ref, w2_ref, b_ref):
    m = m_ref[...].astype(jnp.float32)
    flag = m[:, 0:1] < 1e38
    mx = jnp.where(flag, xd - m, 0.0)
    y = jnp.dot(xd, w1_ref[...], preferred_element_type=jnp.float32)
    y = y + jnp.dot(mx, w2_ref[...], preferred_element_type=jnp.float32)
    y = y + b_ref[...]
    return jnp.maximum(y, 0.01 * y)


def _finalize_body(xd_ref, m_ref, w1_ref, w2_ref, b_ref, o_ref):
    xd = xd_ref[...]
    o_ref[...] = xd + _fin_y(xd, m_ref, w1_ref, w2_ref, b_ref)


def _finalize_body2(xd_ref, m_ref, w1_ref, w2_ref, b_ref, o_ref, o16_ref):
    xd = xd_ref[...]
    o = xd + _fin_y(xd, m_ref, w1_ref, w2_ref, b_ref)
    o_ref[...] = o
    o16_ref[...] = o.astype(jnp.bfloat16)


def _finalize(x_dst, m, W, b, want_bf16=False):
    n = x_dst.shape[0]
    grid = n // ROW_BLK
    blk = pl.BlockSpec((ROW_BLK, OUT), lambda i: (i, 0))
    in_specs = [
        blk,
        blk,
        pl.BlockSpec((OUT, OUT), lambda i: (0, 0)),
        pl.BlockSpec((OUT, OUT), lambda i: (0, 0)),
        pl.BlockSpec((1, OUT), lambda i: (0, 0)),
    ]
    args = (x_dst, m, W[:OUT], W[OUT:], b.reshape(1, OUT))
    if not want_bf16:
        return pl.pallas_call(
            _finalize_body,
            grid=(grid,),
            in_specs=in_specs,
            out_specs=blk,
            out_shape=jax.ShapeDtypeStruct((n, OUT), jnp.float32),
        )(*args)
    return pl.pallas_call(
        _finalize_body2,
        grid=(grid,),
        in_specs=in_specs,
        out_specs=(blk, blk),
        out_shape=(jax.ShapeDtypeStruct((n, OUT), jnp.float32),
                   jax.ShapeDtypeStruct((n, OUT), jnp.bfloat16)),
    )(*args)


@functools.partial(jax.jit, static_argnums=(3,))
def _segmin_sc_call(x_src, src, dst, n_dst):
    m_edges = src.shape[0]
    npass = -(-n_dst // (NW * MAXR))
    R = -(-n_dst // (NW * npass))
    R = -(-R // 8) * 8
    npad = NW * npass * R
    C = EDGE_CHUNK
    nchunk = -(-m_edges // C)
    if nchunk % 2:
        nchunk += 1
    if nchunk * C != m_edges:
        pad = nchunk * C - m_edges
        src = jnp.concatenate([src, jnp.zeros((pad,), jnp.int32)])
        dst = jnp.concatenate([dst, jnp.full((pad,), npad, jnp.int32)])
    npair = nchunk // 2

    mesh = plsc.VectorSubcoreMesh(core_axis_name="c", subcore_axis_name="s")

    def body(xsrc, srch, dsth, outh,
             accf, dstvA, srcvA, dstvB, srcvB, dlocA, sselA, dlocB, sselB,
             idx0, idx1, idx2, idx3, rows0, rows1, rows2, rows3,
             semA, semB, semg0, semg1, semg2, semg3):
        cix = lax.axis_index("c")
        six = lax.axis_index("s")
        wid = six * NC + cix

        def load_chunk(ci, dv, sv, sem):
            pltpu.async_copy(dsth.at[pl.ds(ci * C, C)], dv, sem)
            pltpu.async_copy(srch.at[pl.ds(ci * C, C)], sv, sem)

        def wait_chunk(ci, dv, sv, sem):
            pltpu.make_async_copy(dsth.at[pl.ds(ci * C, C)], dv, sem).wait()
            pltpu.make_async_copy(srch.at[pl.ds(ci * C, C)], sv, sem).wait()

        def start_gather(ssel, j, idxr, rowsr, semg):
            idxr[pl.ds(0, L)] = ssel[pl.ds(j * L, L)]
            pltpu.async_copy(xsrc.at[idxr], rowsr, semg)

        def wait_gather(idxr, rowsr, semg):
            pltpu.make_async_copy(xsrc.at[idxr], rowsr, semg).wait()

        def accum(dloc, j, rowsr):
            dlv = dloc[pl.ds(j * L, L)]
            for e in range(L):
                dle = dlv[e]
                g = rowsr[e, pl.ds(0, OUT)]
                accf[dle, pl.ds(0, OUT)] = jnp.minimum(accf[dle, pl.ds(0, OUT)], g)

        bufs = ((idx0, rows0, semg0), (idx1, rows1, semg1),
                (idx2, rows2, semg2), (idx3, rows3, semg3))

        def pass_body(p, _):
            rowbase = (wid * npass + p) * R
            load_chunk(0, dstvA, srcvA, semA)

            def init_body(i, _):
                big = jnp.full((OUT,), BIG, jnp.bfloat16)
                for q in range(4):
                    accf[4 * i + q, pl.ds(0, OUT)] = big
                return 0

            lax.fori_loop(0, (R + 8) // 4, init_body, 0)

            def scanph(dv, sv, dloc, ssel):
                def group_body(g2, cnt):
                    for u in range(2):
                        g = 2 * g2 + u
                        d = dv[pl.ds(g * L, L)]
                        sidx = sv[pl.ds(g * L, L)]
                        dl = d - rowbase
                        mask = (dl >= 0) & (dl < R)
                        plsc.store_compressed(dloc.at[pl.ds(cnt, L)], dl, mask=mask)
                        plsc.store_compressed(ssel.at[pl.ds(cnt, L)], sidx, mask=mask)
                        cnt = cnt + plsc.all_reduce_population_count(mask)[0]
                    return cnt

                cnt = lax.fori_loop(0, C // L // 2, group_body, 0)
                full_true = jnp.ones((L,), jnp.bool_)
                plsc.store_compressed(dloc.at[pl.ds(cnt, L)],
                                      jnp.full((L,), R, jnp.int32), mask=full_true)
                plsc.store_compressed(ssel.at[pl.ds(cnt, L)],
                                      jnp.zeros((L,), jnp.int32), mask=full_true)
                return (cnt + (L - 1)) // L

            def drain(dloc, ssel, ngr):
                def quad_body(tq, _):
                    for q in range(4):
                        j = 4 * tq + q

                        @pl.when(j < ngr)
                        def _(j=j, q=q):
                            idxr, rowsr, semg = bufs[q]
                            wait_gather(idxr, rowsr, semg)
                            accum(dloc, j, rowsr)

                            @pl.when(j + 4 < ngr)
                            def _(j=j, q=q):
                                start_gather(ssel, j + 4, *bufs[q])

                    return 0

                lax.fori_loop(0, (ngr + 3) // 4, quad_body, 0)

            def fire_first(ssel, ngr):
                for q in range(4):

                    @pl.when(q < ngr)
                    def _(q=q):
                        start_gather(ssel, q, *bufs[q])

            load_chunk(1, dstvB, srcvB, semB)
            wait_chunk(0, dstvA, srcvA, semA)
            ngrA0 = scanph(dstvA, srcvA, dlocA, sselA)
            fire_first(sselA, ngrA0)

            def pipe_body(t, ngrA):
                ci1 = 2 * t + 1
                wait_chunk(ci1, dstvB, srcvB, semB)

                @pl.when(ci1 + 1 < nchunk)
                def _():
                    load_chunk(ci1 + 1, dstvA, srcvA, semA)

                ngrB = scanph(dstvB, srcvB, dlocB, sselB)
                drain(dlocA, sselA, ngrA)
                fire_first(sselB, ngrB)

                @pl.when(ci1 + 2 < nchunk)
                def _():
                    load_chunk(ci1 + 2, dstvB, srcvB, semB)

                def scan_next():
                    wait_chunk(ci1 + 1, dstvA, srcvA, semA)
                    return scanph(dstvA, srcvA, dlocA, sselA)

                ngrA2 = lax.cond(ci1 + 1 < nchunk, scan_next, lambda: 0)
                drain(dlocB, sselB, ngrB)

                @pl.when(ci1 + 1 < nchunk)
                def _():
                    fire_first(sselA, ngrA2)

                return ngrA2

            lax.fori_loop(0, npair, pipe_body, ngrA0)
            pltpu.sync_copy(accf.at[pl.ds(0, R)], outh.at[pl.ds(rowbase, R)])
            return 0

        lax.fori_loop(0, npass, pass_body, 0)

    f = pl.kernel(
        body,
        out_type=jax.ShapeDtypeStruct((npad, OUT), jnp.bfloat16),
        mesh=mesh,
        scratch_types=[
            pltpu.VMEM((R + 8, OUT), jnp.bfloat16),
            pltpu.VMEM((C,), jnp.int32),
            pltpu.VMEM((C,), jnp.int32),
            pltpu.VMEM((C,), jnp.int32),
            pltpu.VMEM((C,), jnp.int32),
            pltpu.VMEM((C + L,), jnp.int32),
            pltpu.VMEM((C + L,), jnp.int32),
            pltpu.VMEM((C + L,), jnp.int32),
            pltpu.VMEM((C + L,), jnp.int32),
            pltpu.VMEM((L,), jnp.int32),
            pltpu.VMEM((L,), jnp.int32),
            pltpu.VMEM((L,), jnp.int32),
            pltpu.VMEM((L,), jnp.int32),
            pltpu.VMEM((L, OUT), jnp.bfloat16),
            pltpu.VMEM((L, OUT), jnp.bfloat16),
            pltpu.VMEM((L, OUT), jnp.bfloat16),
            pltpu.VMEM((L, OUT), jnp.bfloat16),
            pltpu.SemaphoreType.DMA,
            pltpu.SemaphoreType.DMA,
            pltpu.SemaphoreType.DMA,
            pltpu.SemaphoreType.DMA,
            pltpu.SemaphoreType.DMA,
            pltpu.SemaphoreType.DMA,
        ],
        compiler_params=pltpu.CompilerParams(
            needs_layout_passes=False, use_tc_tiling_on_sc=False),
    )
    out = f(x_src, src, dst)
    return out[:n_dst]


def _conv(x_src16, x_dst, e, W, b, want_bf16=False):
    m = _segmin_sc_call(x_src16, e[0], e[1], x_dst.shape[0])
    return _finalize(x_dst, m, W, b, want_bf16=want_bf16)


def kernel(x_f, x_e, x_v, e_fe, e_ev, e_ef, e_ve, Wf, bf, We, be, Wv, bv, Wfe, bfe, Wev, bev):
    x_f, x_f16 = _embed(x_f, Wf, bf, want_bf16=True)
    x_e = _embed(x_e, We, be)
    x_v = _embed(x_v, Wv, bv)
    x_e, x_e16 = _conv(x_f16, x_e, e_fe, Wfe, bfe, want_bf16=True)
    x_v, x_v16 = _conv(x_e16, x_v, e_ev, Wev, bev, want_bf16=True)
    x_f = _conv(x_e16, x_f, e_ef, Wfe, bfe)
    x_e = _conv(x_v16, x_e, e_ve, Wfe, bfe)
    return (x_f, x_e, x_v)
